# baseline (speedup 1.0000x reference)
import jax
import jax.numpy as jnp
from jax import lax
from jax.experimental import pallas as pl
from jax.experimental.pallas import tpu as pltpu
from jax.experimental.pallas import tpu_sc as plsc

B = 16384
D = 64
OUT_COLS = 453

NC = 2
NS = 16
NW = NC * NS
B_PER_W = B // NW
CH = 64
N_CHUNKS = B_PER_W // CH
NG = CH // 16

EMB_COLS = (0, 64, 131, 195, 259, 323, 389)
SCAL = (
    (128, 100.0, 50.0),
    (129, 0.3, 0.1),
    (130, 500.0, 200.0),
    (387, 45.0, 23.0),
    (388, 130.0, 58.0),
)


def _body(i0, i1, i2, i3, i4, i5, i6,
          sales, gmii, visits, price, ml,
          t0, t1, t2, t3, t4, t5, t6,
          out_hbm,
          jv0, jv1, jv2, jv3, jv4, jv5, jv6,
          sv0, sv1, sv2, sv3, sv4,
          g0, g1, g2, g3, g4, g5, g6, asm,
          sem_s, sem_w, s0, s1, s2, s3, s4, s5, s6):
    idx_hbm = (i0, i1, i2, i3, i4, i5, i6)
    tables = (t0, t1, t2, t3, t4, t5, t6)
    jvs = (jv0, jv1, jv2, jv3, jv4, jv5, jv6)
    scal_hbm = (sales, gmii, visits, price, ml)
    svs = (sv0, sv1, sv2, sv3, sv4)
    gbufs = (g0, g1, g2, g3, g4, g5, g6)
    gsems = (s0, s1, s2, s3, s4, s5, s6)

    wid = lax.axis_index("s") * NC + lax.axis_index("c")
    base = wid * B_PER_W
    lane = lax.iota(jnp.int32, 16)
    rows_g = [lane + g * 16 for g in range(NG)]

    def fire_rows(t):
        for g in range(NG):
            vidx = jvs[t][pl.ds(g * 16, 16)]
            for k in range(16):
                r = vidx[k]
                pltpu.async_copy(
                    tables[t].at[pl.ds(r, 1)],
                    gbufs[t].at[pl.ds(g * 16 + k, 1)], gsems[t])

    for t in range(7):
        pltpu.sync_copy(idx_hbm[t].at[pl.ds(base, CH)], jvs[t])
        fire_rows(t)

    def chunk_body(cc, carry):
        rbase = pl.multiple_of(base + cc * CH, CH)
        rnext = pl.multiple_of(rbase + CH, CH)
        last = cc == N_CHUNKS - 1
        ih = []

        @pl.when(jnp.logical_not(last))
        def _():
            for t in range(7):
                ih.append(pltpu.async_copy(
                    idx_hbm[t].at[pl.ds(rnext, CH)], jvs[t], sem_s))
        @pl.when(cc > 0)
        def _():
            pltpu.make_async_copy(
                asm, out_hbm.at[pl.ds(rbase, CH)], sem_w).wait()

        for f, (col, mean, std) in enumerate(SCAL):
            pltpu.sync_copy(scal_hbm[f].at[pl.ds(rbase, CH)], svs[f])
            cols = jnp.full((16,), col, jnp.int32)
            inv = 1.0 / std
            for g in range(NG):
                v = svs[f][pl.ds(g * 16, 16)]
                plsc.store_scatter(asm, [rows_g[g], cols], (v - mean) * inv)

        @pl.when(jnp.logical_not(last))
        def _():
            for h in ih:
                h.wait()

        for t in range(7):
            pltpu.make_async_copy(
                tables[t].at[pl.ds(0, CH)], gbufs[t], gsems[t]).wait()

            def c_body(c, inner):
                csplat = jnp.full((16,), c, jnp.int32)
                dst = csplat + EMB_COLS[t]
                for g in range(NG):
                    v = plsc.load_gather(gbufs[t], [rows_g[g], csplat])
                    plsc.store_scatter(asm, [rows_g[g], dst], v)
                return inner

            lax.fori_loop(0, D, c_body, 0)

            @pl.when(jnp.logical_not(last))
            def _():
                fire_rows(t)

        pltpu.async_copy(asm, out_hbm.at[pl.ds(rbase, CH)], sem_w)
        return carry

    lax.fori_loop(0, N_CHUNKS, chunk_body, 0)
    pltpu.make_async_copy(
        asm, out_hbm.at[pl.ds(base, CH)], sem_w).wait()


@jax.jit
def _sc_call(*args):
    mesh = plsc.VectorSubcoreMesh(core_axis_name="c", subcore_axis_name="s")
    return pl.kernel(
        _body,
        out_type=jax.ShapeDtypeStruct((B, OUT_COLS), jnp.float32),
        mesh=mesh,
        compiler_params=pltpu.CompilerParams(needs_layout_passes=False),
        scratch_types=(
            [pltpu.VMEM((CH,), jnp.int32) for _ in range(7)]
            + [pltpu.VMEM((CH,), jnp.float32) for _ in range(5)]
            + [pltpu.VMEM((CH, D), jnp.float32) for _ in range(7)]
            + [pltpu.VMEM((CH, OUT_COLS), jnp.float32)]
            + [pltpu.SemaphoreType.DMA] * 9
        ),
    )(*args)


def kernel(config_id, brand, category, ptype, series, gender, attributes,
           sales, gmii, visits, price, ml,
           table_product, table_brand, table_category, table_type,
           table_series, table_gender, table_attr):
    idx = [i.astype(jnp.int32)
           for i in (config_id, brand, category, ptype, series, gender,
                     attributes)]
    return _sc_call(*idx, sales, gmii, visits, price, ml,
                    table_product, table_brand, table_category, table_type,
                    table_series, table_gender, table_attr)


# speedup vs baseline: 1.0384x; 1.0384x over previous
import jax
import jax.numpy as jnp
from jax import lax
from jax.experimental import pallas as pl
from jax.experimental.pallas import tpu as pltpu
from jax.experimental.pallas import tpu_sc as plsc

B = 16384
D = 64
OUT_COLS = 453

NC = 2
NS = 16
NW = NC * NS
B_PER_W = B // NW
CH = 64
N_CHUNKS = B_PER_W // CH
NG = CH // 16

EMB_COLS = (0, 64, 131, 195, 259, 323, 389)
SCAL = (
    (128, 100.0, 50.0),
    (129, 0.3, 0.1),
    (130, 500.0, 200.0),
    (387, 45.0, 23.0),
    (388, 130.0, 58.0),
)


def _body(i0, i1, i2, i3, i4, i5, i6,
          sales, gmii, visits, price, ml,
          t0, t1, t2, t3, t4, t5, t6,
          out_hbm,
          jv0, jv1, jv2, jv3, jv4, jv5, jv6,
          sv0, sv1, sv2, sv3, sv4,
          g0, g1, g2, g3, g4, g5, g6, asm,
          sem_s, sem_w, s0, s1, s2, s3, s4, s5, s6):
    idx_hbm = (i0, i1, i2, i3, i4, i5, i6)
    tables = (t0, t1, t2, t3, t4, t5, t6)
    jvs = (jv0, jv1, jv2, jv3, jv4, jv5, jv6)
    scal_hbm = (sales, gmii, visits, price, ml)
    svs = (sv0, sv1, sv2, sv3, sv4)
    gbufs = (g0, g1, g2, g3, g4, g5, g6)
    gsems = (s0, s1, s2, s3, s4, s5, s6)

    wid = lax.axis_index("s") * NC + lax.axis_index("c")
    base = wid * B_PER_W
    lane = lax.iota(jnp.int32, 16)
    rows_g = [lane + g * 16 for g in range(NG)]

    def chunk_body(cc, carry):
        rbase = pl.multiple_of(base + cc * CH, CH)
        hs = []
        for src, dst in zip(idx_hbm + scal_hbm, jvs + svs):
            hs.append(pltpu.async_copy(src.at[pl.ds(rbase, CH)], dst, sem_s))
        @pl.when(cc > 0)
        def _():
            pltpu.make_async_copy(
                asm, out_hbm.at[pl.ds(rbase, CH)], sem_w).wait()
        for h in hs:
            h.wait()

        for t in range(7):
            for g in range(NG):
                vidx = jvs[t][pl.ds(g * 16, 16)]
                for k in range(16):
                    r = vidx[k]
                    pltpu.async_copy(
                        tables[t].at[pl.ds(r, 1)],
                        gbufs[t].at[pl.ds(g * 16 + k, 1)], gsems[t])

        for f, (col, mean, std) in enumerate(SCAL):
            cols = jnp.full((16,), col, jnp.int32)
            inv = 1.0 / std
            for g in range(NG):
                v = svs[f][pl.ds(g * 16, 16)]
                plsc.store_scatter(asm, [rows_g[g], cols], (v - mean) * inv)

        for t in range(7):
            pltpu.make_async_copy(
                tables[t].at[pl.ds(0, CH)], gbufs[t], gsems[t]).wait()

            def c_body(c, inner):
                csplat = jnp.full((16,), c, jnp.int32)
                dst = csplat + EMB_COLS[t]
                for g in range(NG):
                    v = plsc.load_gather(gbufs[t], [rows_g[g], csplat])
                    plsc.store_scatter(asm, [rows_g[g], dst], v)
                return inner

            lax.fori_loop(0, D, c_body, 0)
        pltpu.async_copy(asm, out_hbm.at[pl.ds(rbase, CH)], sem_w)
        return carry

    lax.fori_loop(0, N_CHUNKS, chunk_body, 0)
    pltpu.make_async_copy(
        asm, out_hbm.at[pl.ds(base, CH)], sem_w).wait()


@jax.jit
def _sc_call(*args):
    mesh = plsc.VectorSubcoreMesh(core_axis_name="c", subcore_axis_name="s")
    return pl.kernel(
        _body,
        out_type=jax.ShapeDtypeStruct((B, OUT_COLS), jnp.float32),
        mesh=mesh,
        compiler_params=pltpu.CompilerParams(needs_layout_passes=False),
        scratch_types=(
            [pltpu.VMEM((CH,), jnp.int32) for _ in range(7)]
            + [pltpu.VMEM((CH,), jnp.float32) for _ in range(5)]
            + [pltpu.VMEM((CH, D), jnp.float32) for _ in range(7)]
            + [pltpu.VMEM((CH, OUT_COLS), jnp.float32)]
            + [pltpu.SemaphoreType.DMA] * 9
        ),
    )(*args)


def kernel(config_id, brand, category, ptype, series, gender, attributes,
           sales, gmii, visits, price, ml,
           table_product, table_brand, table_category, table_type,
           table_series, table_gender, table_attr):
    idx = [i.astype(jnp.int32)
           for i in (config_id, brand, category, ptype, series, gender,
                     attributes)]
    return _sc_call(*idx, sales, gmii, visits, price, ml,
                    table_product, table_brand, table_category, table_type,
                    table_series, table_gender, table_attr)
